# initial kernel scaffold (unmeasured)
import jax
import jax.numpy as jnp
from jax import lax
from jax.experimental import pallas as pl
from jax.experimental.pallas import tpu as pltpu

N_DEV = 8


def kernel(x, w_mat):
    m_loc, k = x.shape
    n = w_mat.shape[1]
    n_loc = n // N_DEV
    m_glob = m_loc * N_DEV

    def body(x_ref, w_ref, out_ref, y_ref, send_buf, recv_buf,
             amax_send, amax_recv, send_sems, recv_sems,
             amax_send_sems, amax_recv_sems):
        my = lax.axis_index("i")

        y_ref[:, :] = jnp.dot(x_ref[:, :], w_ref[:, :],
                              preferred_element_type=jnp.float32)
        my_amax = jnp.max(jnp.abs(y_ref[:, :]))
        amax_send[:, :] = jnp.broadcast_to(my_amax, (8, 128))

        chunk_rdmas = []
        amax_rdmas = []
        for d in range(1, N_DEV):
            j = lax.rem(my + d, N_DEV)
            send_buf[d, :, :] = y_ref[:, pl.ds(j * n_loc, n_loc)].astype(
                jnp.bfloat16)
            r = pltpu.make_async_remote_copy(
                src_ref=send_buf.at[d],
                dst_ref=recv_buf.at[d],
                send_sem=send_sems.at[d],
                recv_sem=recv_sems.at[d],
                device_id=(j,),
                device_id_type=pl.DeviceIdType.MESH,
            )
            r.start()
            chunk_rdmas.append(r)
            ra = pltpu.make_async_remote_copy(
                src_ref=amax_send,
                dst_ref=amax_recv.at[d],
                send_sem=amax_send_sems.at[d],
                recv_sem=amax_recv_sems.at[d],
                device_id=(j,),
                device_id_type=pl.DeviceIdType.MESH,
            )
            ra.start()
            amax_rdmas.append(ra)

        g_amax = my_amax
        for d in range(1, N_DEV):
            amax_rdmas[d - 1].wait()
            g_amax = jnp.maximum(g_amax, amax_recv[d, 0, 0])
        scale = g_amax / 448.0
        inv_scale = 448.0 / g_amax

        def qdq(v):
            q = jnp.clip(v * inv_scale, -448.0, 448.0)
            q = q.astype(jnp.float8_e4m3fn).astype(jnp.float32)
            return q * scale

        out_ref[pl.ds(my * m_loc, m_loc), :] = qdq(
            y_ref[:, pl.ds(my * n_loc, n_loc)])

        for d in range(1, N_DEV):
            chunk_rdmas[d - 1].wait()
            s = lax.rem(my - d + N_DEV, N_DEV)
            out_ref[pl.ds(s * m_loc, m_loc), :] = qdq(
                recv_buf[d, :, :].astype(jnp.float32))

    return pl.pallas_call(
        body,
        out_shape=jax.ShapeDtypeStruct((m_glob, n_loc), jnp.float32),
        in_specs=[pl.BlockSpec(memory_space=pltpu.VMEM),
                  pl.BlockSpec(memory_space=pltpu.VMEM)],
        out_specs=pl.BlockSpec(memory_space=pltpu.VMEM),
        scratch_shapes=[
            pltpu.VMEM((m_loc, n), jnp.float32),
            pltpu.VMEM((N_DEV, m_loc, n_loc), jnp.bfloat16),
            pltpu.VMEM((N_DEV, m_loc, n_loc), jnp.bfloat16),
            pltpu.VMEM((8, 128), jnp.float32),
            pltpu.VMEM((N_DEV, 8, 128), jnp.float32),
            pltpu.SemaphoreType.DMA((N_DEV,)),
            pltpu.SemaphoreType.DMA((N_DEV,)),
            pltpu.SemaphoreType.DMA((N_DEV,)),
            pltpu.SemaphoreType.DMA((N_DEV,)),
        ],
        compiler_params=pltpu.CompilerParams(collective_id=0),
    )(x, w_mat)


# baseline (device time: 53069 ns/iter reference)
import jax
import jax.numpy as jnp
from jax import lax
from jax.experimental import pallas as pl
from jax.experimental.pallas import tpu as pltpu

N_DEV = 8


def kernel(x, w_mat):
    m_loc, k = x.shape
    n = w_mat.shape[1]
    n_loc = n // N_DEV
    m_glob = m_loc * N_DEV

    def body(x_ref, w_ref, out_ref, y_ref, send_buf, recv_buf,
             amax_send, amax_recv, send_sems, recv_sems,
             amax_send_sems, amax_recv_sems):
        my = lax.axis_index("i")

        y_ref[:, :] = jnp.dot(x_ref[:, :].astype(jnp.bfloat16),
                              w_ref[:, :].astype(jnp.bfloat16),
                              preferred_element_type=jnp.float32)
        my_amax = jnp.max(jnp.abs(y_ref[:, :]))
        amax_send[:, :] = jnp.broadcast_to(my_amax, (8, 128))

        chunk_rdmas = []
        amax_rdmas = []
        for d in range(1, N_DEV):
            j = lax.rem(my + d, N_DEV)
            send_buf[d, :, :] = y_ref[:, pl.ds(j * n_loc, n_loc)].astype(
                jnp.bfloat16)
            r = pltpu.make_async_remote_copy(
                src_ref=send_buf.at[d],
                dst_ref=recv_buf.at[d],
                send_sem=send_sems.at[d],
                recv_sem=recv_sems.at[d],
                device_id=(j,),
                device_id_type=pl.DeviceIdType.MESH,
            )
            r.start()
            chunk_rdmas.append(r)
            ra = pltpu.make_async_remote_copy(
                src_ref=amax_send,
                dst_ref=amax_recv.at[d],
                send_sem=amax_send_sems.at[d],
                recv_sem=amax_recv_sems.at[d],
                device_id=(j,),
                device_id_type=pl.DeviceIdType.MESH,
            )
            ra.start()
            amax_rdmas.append(ra)

        g_amax = my_amax
        for d in range(1, N_DEV):
            amax_rdmas[d - 1].wait()
            g_amax = jnp.maximum(g_amax, amax_recv[d, 0, 0])
        scale = g_amax / 448.0
        inv_scale = 448.0 / g_amax

        def qdq(v):
            q = jnp.clip(v * inv_scale, -448.0, 448.0)
            q = q.astype(jnp.float8_e4m3fn).astype(jnp.float32)
            return q * scale

        out_ref[pl.ds(my * m_loc, m_loc), :] = qdq(
            y_ref[:, pl.ds(my * n_loc, n_loc)])

        for d in range(1, N_DEV):
            chunk_rdmas[d - 1].wait()
            s = lax.rem(my - d + N_DEV, N_DEV)
            out_ref[pl.ds(s * m_loc, m_loc), :] = qdq(
                recv_buf[d, :, :].astype(jnp.float32))

    return pl.pallas_call(
        body,
        out_shape=jax.ShapeDtypeStruct((m_glob, n_loc), jnp.float32),
        in_specs=[pl.BlockSpec(memory_space=pltpu.VMEM),
                  pl.BlockSpec(memory_space=pltpu.VMEM)],
        out_specs=pl.BlockSpec(memory_space=pltpu.VMEM),
        scratch_shapes=[
            pltpu.VMEM((m_loc, n), jnp.float32),
            pltpu.VMEM((N_DEV, m_loc, n_loc), jnp.bfloat16),
            pltpu.VMEM((N_DEV, m_loc, n_loc), jnp.bfloat16),
            pltpu.VMEM((8, 128), jnp.float32),
            pltpu.VMEM((N_DEV, 8, 128), jnp.float32),
            pltpu.SemaphoreType.DMA((N_DEV,)),
            pltpu.SemaphoreType.DMA((N_DEV,)),
            pltpu.SemaphoreType.DMA((N_DEV,)),
            pltpu.SemaphoreType.DMA((N_DEV,)),
        ],
        compiler_params=pltpu.CompilerParams(
            vmem_limit_bytes=128 * 1024 * 1024),
    )(x, w_mat)


# device time: 46187 ns/iter; 1.1490x vs baseline; 1.1490x over previous
import jax
import jax.numpy as jnp
from jax import lax
from jax.experimental import pallas as pl
from jax.experimental.pallas import tpu as pltpu

N_DEV = 8
F8 = jnp.float8_e4m3fn


def kernel(x, w_mat):
    m_loc, k = x.shape
    n = w_mat.shape[1]
    n_loc = n // N_DEV
    m_glob = m_loc * N_DEV

    def body(x_ref, w_ref, out_ref, y_buf, send_q, recv_q,
             amax_send, amax_recv, send_sems, recv_sems,
             amax_send_sems, amax_recv_sems):
        my = lax.axis_index("i")

        bsem = pltpu.get_barrier_semaphore()
        for p in range(N_DEV):
            @pl.when(p != my)
            def _(p=p):
                pl.semaphore_signal(bsem, inc=1, device_id=(p,),
                                    device_id_type=pl.DeviceIdType.MESH)

        amaxes = []
        for d in list(range(1, N_DEV)) + [0]:
            jj = lax.rem(my + d, N_DEV)
            yc = jnp.dot(x_ref[:, :], w_ref[:, pl.ds(jj * n_loc, n_loc)],
                         preferred_element_type=jnp.float32)
            if d == 1:
                pl.semaphore_wait(bsem, N_DEV - 1)
            amaxes.append(jnp.max(jnp.abs(yc)))
            y_buf[d, :, :] = yc

        my_amax = amaxes[0]
        for a in amaxes[1:]:
            my_amax = jnp.maximum(my_amax, a)
        amax_send[:, :] = jnp.broadcast_to(my_amax, (8, 128))

        amax_rdmas = []
        for c in range(N_DEV):
            ra = pltpu.make_async_remote_copy(
                src_ref=amax_send,
                dst_ref=amax_recv.at[my],
                send_sem=amax_send_sems.at[c],
                recv_sem=amax_recv_sems.at[my],
                device_id=(c,),
                device_id_type=pl.DeviceIdType.MESH,
            )
            amax_rdmas.append(ra)

            @pl.when(c != my)
            def _(ra=ra):
                ra.start()

        for s in range(N_DEV):
            raw = pltpu.make_async_remote_copy(
                src_ref=amax_send,
                dst_ref=amax_recv.at[s],
                send_sem=amax_send_sems.at[s],
                recv_sem=amax_recv_sems.at[s],
                device_id=(s,),
                device_id_type=pl.DeviceIdType.MESH,
            )

            @pl.when(s != my)
            def _(raw=raw):
                raw.wait_recv()

        g_amax = my_amax
        for s in range(N_DEV):
            g_amax = jnp.maximum(
                g_amax,
                jnp.where(s == my, my_amax, amax_recv[s, 0, 0]))
        scale = g_amax / 448.0
        inv_scale = 448.0 / g_amax

        def quant(v):
            return jnp.clip(v * inv_scale, -448.0, 448.0).astype(F8)

        chunk_rdmas = []
        for d in range(1, N_DEV):
            jj = lax.rem(my + d, N_DEV)
            send_q[d, :, :] = quant(y_buf[d, :, :])
            r = pltpu.make_async_remote_copy(
                src_ref=send_q.at[d],
                dst_ref=recv_q.at[my],
                send_sem=send_sems.at[d],
                recv_sem=recv_sems.at[my],
                device_id=(jj,),
                device_id_type=pl.DeviceIdType.MESH,
            )
            r.start()
            chunk_rdmas.append(r)

        for s in range(N_DEV):
            @pl.when(s == my)
            def _(s=s):
                out_ref[s * m_loc:(s + 1) * m_loc, :] = (
                    quant(y_buf[0, :, :]).astype(jnp.float32) * scale)

        for s in range(N_DEV):
            rw = pltpu.make_async_remote_copy(
                src_ref=send_q.at[s],
                dst_ref=recv_q.at[s],
                send_sem=send_sems.at[s],
                recv_sem=recv_sems.at[s],
                device_id=(s,),
                device_id_type=pl.DeviceIdType.MESH,
            )

            @pl.when(s != my)
            def _(rw=rw, s=s):
                rw.wait_recv()
                out_ref[s * m_loc:(s + 1) * m_loc, :] = (
                    recv_q[s, :, :].astype(jnp.float32) * scale)

        for r in chunk_rdmas:
            r.wait_send()
        for c in range(N_DEV):
            @pl.when(c != my)
            def _(c=c):
                amax_rdmas[c].wait_send()

    return pl.pallas_call(
        body,
        out_shape=jax.ShapeDtypeStruct((m_glob, n_loc), jnp.float32),
        in_specs=[pl.BlockSpec(memory_space=pltpu.VMEM),
                  pl.BlockSpec(memory_space=pltpu.VMEM)],
        out_specs=pl.BlockSpec(memory_space=pltpu.VMEM),
        scratch_shapes=[
            pltpu.VMEM((N_DEV, m_loc, n_loc), jnp.float32),
            pltpu.VMEM((N_DEV, m_loc, n_loc), F8),
            pltpu.VMEM((N_DEV, m_loc, n_loc), F8),
            pltpu.VMEM((8, 128), jnp.float32),
            pltpu.VMEM((N_DEV, 8, 128), jnp.float32),
            pltpu.SemaphoreType.DMA((N_DEV,)),
            pltpu.SemaphoreType.DMA((N_DEV,)),
            pltpu.SemaphoreType.DMA((N_DEV,)),
            pltpu.SemaphoreType.DMA((N_DEV,)),
        ],
        compiler_params=pltpu.CompilerParams(
            vmem_limit_bytes=128 * 1024 * 1024, collective_id=0),
    )(x, w_mat)
